# Initial kernel scaffold; baseline (speedup 1.0000x reference)
import jax
import jax.numpy as jnp
from jax.experimental import pallas as pl


def kernel(x, edge_index, W1, b1, W2, b2):
    raise NotImplementedError("write your pallas kernel here")


# speedup vs baseline: 21.4238x; 21.4238x over previous
import functools

import jax
import jax.numpy as jnp
from jax import lax
from jax.experimental import pallas as pl
from jax.experimental.pallas import tpu as pltpu
from jax.experimental.pallas import tpu_sc as plsc

N = 10000
E = 320000
D_IN = 128
D_HID = 128
D_OUT = 64

NC = 2
NS = 16
NW = NC * NS
EPW = E // NW
K = 80
NCHUNK = EPW // K
RPT = 624
TAIL = N - NS * RPT
ZR = 24

_MESH = plsc.VectorSubcoreMesh(core_axis_name="c", subcore_axis_name="s")


def _fill_rows(ref, rows, d, value):
    splat = jnp.full((16,), value, dtype=jnp.float32)

    def body(i, carry):
        for jcol in range(d // 16):
            ref[i, pl.ds(jcol * 16, 16)] = splat
        return carry

    lax.fori_loop(0, rows, body, 0)


def _make_agg(d):
    @functools.partial(
        pl.kernel,
        out_type=jax.ShapeDtypeStruct((NC, N, d), jnp.float32),
        mesh=_MESH,
        compiler_params=pltpu.CompilerParams(use_tc_tiling_on_sc=False),
        scratch_types=[
            pltpu.VMEM((NCHUNK, K), jnp.int32),
            pltpu.VMEM((NCHUNK, K), jnp.int32),
            pltpu.VMEM((K, d), jnp.float32),
            pltpu.VMEM((ZR, d), jnp.float32),
            pltpu.VMEM_SHARED((N, d), jnp.float32),
        ],
    )
    def agg(g_hbm, src_hbm, dst_hbm, out_hbm, src_v, dst_v, rows_v, zero_v, acc):
        c = lax.axis_index("c")
        s = lax.axis_index("s")
        wid = c * NS + s

        _fill_rows(zero_v, ZR, d, 0.0)
        for z in range(RPT // ZR):
            pltpu.sync_copy(zero_v, acc.at[pl.ds(s * RPT + z * ZR, ZR)])

        @pl.when(s == 0)
        def _():
            pltpu.sync_copy(zero_v.at[pl.ds(0, TAIL)],
                            acc.at[pl.ds(NS * RPT, TAIL)])

        plsc.subcore_barrier()

        pltpu.sync_copy(src_hbm.at[wid], src_v)
        pltpu.sync_copy(dst_hbm.at[wid], dst_v)

        def chunk(j, carry):
            pltpu.sync_copy(g_hbm.at[src_v.at[j]], rows_v)
            pltpu.sync_copy(rows_v, acc.at[dst_v.at[j]], add=True)
            return carry

        lax.fori_loop(0, NCHUNK, chunk, 0)
        plsc.subcore_barrier()

        pltpu.sync_copy(acc.at[pl.ds(s * RPT, RPT)],
                        out_hbm.at[c, pl.ds(s * RPT, RPT)])

        @pl.when(s == 0)
        def _():
            pltpu.sync_copy(acc.at[pl.ds(NS * RPT, TAIL)],
                            out_hbm.at[c, pl.ds(NS * RPT, TAIL)])

    return agg


_agg_hid = _make_agg(D_HID)
_agg_out = _make_agg(D_OUT)

_DEG_W = 16


@functools.partial(
    pl.kernel,
    out_type=jax.ShapeDtypeStruct((NC, N, _DEG_W), jnp.float32),
    mesh=_MESH,
    compiler_params=pltpu.CompilerParams(use_tc_tiling_on_sc=False),
    scratch_types=[
        pltpu.VMEM((NCHUNK, K), jnp.int32),
        pltpu.VMEM((K, _DEG_W), jnp.float32),
        pltpu.VMEM((ZR, _DEG_W), jnp.float32),
        pltpu.VMEM_SHARED((N, _DEG_W), jnp.float32),
    ],
)
def _deg_kernel(dst_hbm, out_hbm, dst_v, ones_v, zero_v, acc):
    c = lax.axis_index("c")
    s = lax.axis_index("s")
    wid = c * NS + s

    _fill_rows(zero_v, ZR, _DEG_W, 0.0)
    _fill_rows(ones_v, K, _DEG_W, 1.0)
    for z in range(RPT // ZR):
        pltpu.sync_copy(zero_v, acc.at[pl.ds(s * RPT + z * ZR, ZR)])

    @pl.when(s == 0)
    def _():
        pltpu.sync_copy(zero_v.at[pl.ds(0, TAIL)],
                        acc.at[pl.ds(NS * RPT, TAIL)])

    plsc.subcore_barrier()

    pltpu.sync_copy(dst_hbm.at[wid], dst_v)

    def chunk(j, carry):
        pltpu.sync_copy(ones_v, acc.at[dst_v.at[j]], add=True)
        return carry

    lax.fori_loop(0, NCHUNK, chunk, 0)
    plsc.subcore_barrier()

    pltpu.sync_copy(acc.at[pl.ds(s * RPT, RPT)],
                    out_hbm.at[c, pl.ds(s * RPT, RPT)])

    @pl.when(s == 0)
    def _():
        pltpu.sync_copy(acc.at[pl.ds(NS * RPT, TAIL)],
                        out_hbm.at[c, pl.ds(NS * RPT, TAIL)])


BLK = 1000


def _tc_a_body(x_ref, w1_ref, degp_ref, g_ref, dinv_ref):
    deg = degp_ref[0, :, 0:1] + degp_ref[1, :, 0:1] + 1.0
    dinv = lax.rsqrt(deg)
    h = jnp.dot(x_ref[...], w1_ref[...], preferred_element_type=jnp.float32)
    g_ref[...] = h * dinv
    dinv_ref[...] = jnp.broadcast_to(dinv, (BLK, _DEG_W))


def _tc_b_body(p_ref, g1_ref, dinv_ref, b1_ref, w2_ref, g2_ref):
    dinv = dinv_ref[:, 0:1]
    x2 = jnp.maximum((p_ref[0] + p_ref[1] + g1_ref[...]) * dinv + b1_ref[...],
                     0.0)
    g2_ref[...] = jnp.dot(x2, w2_ref[...],
                          preferred_element_type=jnp.float32) * dinv


def _tc_c_body(q_ref, g2_ref, dinv_ref, b2_ref, out_ref):
    dinv = dinv_ref[:, 0:1]
    z = (q_ref[0] + q_ref[1] + g2_ref[...]) * dinv + b2_ref[...]
    z = z - jnp.max(z, axis=1, keepdims=True)
    ez = jnp.exp(z)
    out_ref[...] = ez / jnp.sum(ez, axis=1, keepdims=True)


def _row_blocked(width):
    return pl.BlockSpec((BLK, width), lambda i: (i, 0))


def _pair_blocked(width):
    return pl.BlockSpec((2, BLK, width), lambda i: (0, i, 0))


def _full(shape):
    return pl.BlockSpec(shape, lambda i: tuple(0 for _ in shape))


def kernel(x, edge_index, W1, b1, W2, b2):
    src = edge_index[0].astype(jnp.int32).reshape(NW, NCHUNK, K)
    dst = edge_index[1].astype(jnp.int32).reshape(NW, NCHUNK, K)

    degp = _deg_kernel(dst)

    grid = N // BLK
    g1, dinv16 = pl.pallas_call(
        _tc_a_body,
        grid=(grid,),
        in_specs=[_row_blocked(D_IN), _full((D_IN, D_HID)),
                  _pair_blocked(_DEG_W)],
        out_specs=[_row_blocked(D_HID), _row_blocked(_DEG_W)],
        out_shape=[jax.ShapeDtypeStruct((N, D_HID), jnp.float32),
                   jax.ShapeDtypeStruct((N, _DEG_W), jnp.float32)],
    )(x, W1, degp)

    p = _agg_hid(g1, src, dst)

    g2 = pl.pallas_call(
        _tc_b_body,
        grid=(grid,),
        in_specs=[_pair_blocked(D_HID), _row_blocked(D_HID),
                  _row_blocked(_DEG_W), _full((1, D_HID)),
                  _full((D_HID, D_OUT))],
        out_specs=_row_blocked(D_OUT),
        out_shape=jax.ShapeDtypeStruct((N, D_OUT), jnp.float32),
    )(p, g1, dinv16, b1.reshape(1, D_HID), W2)

    q = _agg_out(g2, src, dst)

    out = pl.pallas_call(
        _tc_c_body,
        grid=(grid,),
        in_specs=[_pair_blocked(D_OUT), _row_blocked(D_OUT),
                  _row_blocked(_DEG_W), _full((1, D_OUT))],
        out_specs=_row_blocked(D_OUT),
        out_shape=jax.ShapeDtypeStruct((N, D_OUT), jnp.float32),
    )(q, g2, dinv16, b2.reshape(1, D_OUT))

    return out


# speedup vs baseline: 25.2433x; 1.1783x over previous
import functools

import jax
import jax.numpy as jnp
from jax import lax
from jax.experimental import pallas as pl
from jax.experimental.pallas import tpu as pltpu
from jax.experimental.pallas import tpu_sc as plsc

N = 10000
E = 320000
D_IN = 128
D_HID = 128
D_OUT = 64

NC = 2
NS = 16
NW = NC * NS
EPW = E // NW
K = 40
NCHUNK = EPW // K
RPT = 624
TAIL = N - NS * RPT
ZR = 24

_MESH = plsc.VectorSubcoreMesh(core_axis_name="c", subcore_axis_name="s")


def _fill_rows(ref, rows, d, value):
    splat = jnp.full((16,), value, dtype=jnp.float32)

    def body(i, carry):
        for jcol in range(d // 16):
            ref[i, pl.ds(jcol * 16, 16)] = splat
        return carry

    lax.fori_loop(0, rows, body, 0)


def _make_agg(d):
    @functools.partial(
        pl.kernel,
        out_type=jax.ShapeDtypeStruct((NC, N, d), jnp.float32),
        mesh=_MESH,
        compiler_params=pltpu.CompilerParams(use_tc_tiling_on_sc=False),
        scratch_types=[
            pltpu.VMEM((NCHUNK, K), jnp.int32),
            pltpu.VMEM((NCHUNK, K), jnp.int32),
            pltpu.VMEM((2, K, d), jnp.float32),
            pltpu.VMEM((ZR, d), jnp.float32),
            pltpu.VMEM_SHARED((N, d), jnp.float32),
            pltpu.SemaphoreType.DMA,
            pltpu.SemaphoreType.DMA,
            pltpu.SemaphoreType.DMA,
        ],
    )
    def agg(g_hbm, src_hbm, dst_hbm, out_hbm, src_v, dst_v, rows_v, zero_v,
            acc, gsem0, gsem1, zsem):
        c = lax.axis_index("c")
        s = lax.axis_index("s")
        wid = c * NS + s

        _fill_rows(zero_v, ZR, d, 0.0)

        def zcopy(z):
            return pltpu.make_async_copy(
                zero_v, acc.at[pl.ds(s * RPT + z * ZR, ZR)], zsem)

        def zbody(z, carry):
            zcopy(z).start()
            return carry

        lax.fori_loop(0, RPT // ZR, zbody, 0)

        @pl.when(s == 0)
        def _():
            pltpu.sync_copy(zero_v.at[pl.ds(0, TAIL)],
                            acc.at[pl.ds(NS * RPT, TAIL)])

        pltpu.sync_copy(src_hbm.at[wid], src_v)
        pltpu.sync_copy(dst_hbm.at[wid], dst_v)

        def zdrain(z, carry):
            zcopy(z).wait()
            return carry

        lax.fori_loop(0, RPT // ZR, zdrain, 0)
        plsc.subcore_barrier()

        def gather(j, buf, sem):
            return pltpu.make_async_copy(
                g_hbm.at[src_v.at[j]], rows_v.at[buf], sem)

        def scatter(j, buf):
            pltpu.sync_copy(rows_v.at[buf], acc.at[dst_v.at[j]], add=True)

        gather(0, 0, gsem0).start()
        gather(1, 1, gsem1).start()

        def pair(i, carry):
            j0 = 2 * i
            gather(j0, 0, gsem0).wait()
            scatter(j0, 0)

            @pl.when(j0 + 2 < NCHUNK)
            def _():
                gather(j0 + 2, 0, gsem0).start()

            gather(j0 + 1, 1, gsem1).wait()
            scatter(j0 + 1, 1)

            @pl.when(j0 + 3 < NCHUNK)
            def _():
                gather(j0 + 3, 1, gsem1).start()

            return carry

        lax.fori_loop(0, NCHUNK // 2, pair, 0)
        plsc.subcore_barrier()

        pltpu.sync_copy(acc.at[pl.ds(s * RPT, RPT)],
                        out_hbm.at[c, pl.ds(s * RPT, RPT)])

        @pl.when(s == 0)
        def _():
            pltpu.sync_copy(acc.at[pl.ds(NS * RPT, TAIL)],
                            out_hbm.at[c, pl.ds(NS * RPT, TAIL)])

    return agg


_agg_hid = _make_agg(D_HID)
_agg_out = _make_agg(D_OUT)

_DEG_W = 16


@functools.partial(
    pl.kernel,
    out_type=jax.ShapeDtypeStruct((NC, N, _DEG_W), jnp.float32),
    mesh=_MESH,
    compiler_params=pltpu.CompilerParams(use_tc_tiling_on_sc=False),
    scratch_types=[
        pltpu.VMEM((NCHUNK, K), jnp.int32),
        pltpu.VMEM((K, _DEG_W), jnp.float32),
        pltpu.VMEM((ZR, _DEG_W), jnp.float32),
        pltpu.VMEM_SHARED((N, _DEG_W), jnp.float32),
    ],
)
def _deg_kernel(dst_hbm, out_hbm, dst_v, ones_v, zero_v, acc):
    c = lax.axis_index("c")
    s = lax.axis_index("s")
    wid = c * NS + s

    _fill_rows(zero_v, ZR, _DEG_W, 0.0)
    _fill_rows(ones_v, K, _DEG_W, 1.0)
    for z in range(RPT // ZR):
        pltpu.sync_copy(zero_v, acc.at[pl.ds(s * RPT + z * ZR, ZR)])

    @pl.when(s == 0)
    def _():
        pltpu.sync_copy(zero_v.at[pl.ds(0, TAIL)],
                        acc.at[pl.ds(NS * RPT, TAIL)])

    plsc.subcore_barrier()

    pltpu.sync_copy(dst_hbm.at[wid], dst_v)

    def chunk(j, carry):
        pltpu.sync_copy(ones_v, acc.at[dst_v.at[j]], add=True)
        return carry

    lax.fori_loop(0, NCHUNK, chunk, 0)
    plsc.subcore_barrier()

    pltpu.sync_copy(acc.at[pl.ds(s * RPT, RPT)],
                    out_hbm.at[c, pl.ds(s * RPT, RPT)])

    @pl.when(s == 0)
    def _():
        pltpu.sync_copy(acc.at[pl.ds(NS * RPT, TAIL)],
                        out_hbm.at[c, pl.ds(NS * RPT, TAIL)])


BLK = 1000


def _tc_a_body(x_ref, w1_ref, degp_ref, g_ref, dinv_ref):
    deg = degp_ref[0, :, 0:1] + degp_ref[1, :, 0:1] + 1.0
    dinv = lax.rsqrt(deg)
    h = jnp.dot(x_ref[...], w1_ref[...], preferred_element_type=jnp.float32)
    g_ref[...] = h * dinv
    dinv_ref[...] = jnp.broadcast_to(dinv, (BLK, _DEG_W))


def _tc_b_body(p_ref, g1_ref, dinv_ref, b1_ref, w2_ref, g2_ref):
    dinv = dinv_ref[:, 0:1]
    x2 = jnp.maximum((p_ref[0] + p_ref[1] + g1_ref[...]) * dinv + b1_ref[...],
                     0.0)
    g2_ref[...] = jnp.dot(x2, w2_ref[...],
                          preferred_element_type=jnp.float32) * dinv


def _tc_c_body(q_ref, g2_ref, dinv_ref, b2_ref, out_ref):
    dinv = dinv_ref[:, 0:1]
    z = (q_ref[0] + q_ref[1] + g2_ref[...]) * dinv + b2_ref[...]
    z = z - jnp.max(z, axis=1, keepdims=True)
    ez = jnp.exp(z)
    out_ref[...] = ez / jnp.sum(ez, axis=1, keepdims=True)


def _row_blocked(width):
    return pl.BlockSpec((BLK, width), lambda i: (i, 0))


def _pair_blocked(width):
    return pl.BlockSpec((2, BLK, width), lambda i: (0, i, 0))


def _full(shape):
    return pl.BlockSpec(shape, lambda i: tuple(0 for _ in shape))


def kernel(x, edge_index, W1, b1, W2, b2):
    src = edge_index[0].astype(jnp.int32).reshape(NW, NCHUNK, K)
    dst = edge_index[1].astype(jnp.int32).reshape(NW, NCHUNK, K)

    degp = _deg_kernel(dst)

    grid = N // BLK
    g1, dinv16 = pl.pallas_call(
        _tc_a_body,
        grid=(grid,),
        in_specs=[_row_blocked(D_IN), _full((D_IN, D_HID)),
                  _pair_blocked(_DEG_W)],
        out_specs=[_row_blocked(D_HID), _row_blocked(_DEG_W)],
        out_shape=[jax.ShapeDtypeStruct((N, D_HID), jnp.float32),
                   jax.ShapeDtypeStruct((N, _DEG_W), jnp.float32)],
    )(x, W1, degp)

    p = _agg_hid(g1, src, dst)

    g2 = pl.pallas_call(
        _tc_b_body,
        grid=(grid,),
        in_specs=[_pair_blocked(D_HID), _row_blocked(D_HID),
                  _row_blocked(_DEG_W), _full((1, D_HID)),
                  _full((D_HID, D_OUT))],
        out_specs=_row_blocked(D_OUT),
        out_shape=jax.ShapeDtypeStruct((N, D_OUT), jnp.float32),
    )(p, g1, dinv16, b1.reshape(1, D_HID), W2)

    q = _agg_out(g2, src, dst)

    out = pl.pallas_call(
        _tc_c_body,
        grid=(grid,),
        in_specs=[_pair_blocked(D_OUT), _row_blocked(D_OUT),
                  _row_blocked(_DEG_W), _full((1, D_OUT))],
        out_specs=_row_blocked(D_OUT),
        out_shape=jax.ShapeDtypeStruct((N, D_OUT), jnp.float32),
    )(q, g2, dinv16, b2.reshape(1, D_OUT))

    return out


# speedup vs baseline: 26.5046x; 1.0500x over previous
import functools

import jax
import jax.numpy as jnp
from jax import lax
from jax.experimental import pallas as pl
from jax.experimental.pallas import tpu as pltpu
from jax.experimental.pallas import tpu_sc as plsc

N = 10000
E = 320000
D_IN = 128
D_HID = 128
D_OUT = 64

NC = 2
NS = 16
NW = NC * NS
EPW = E // NW
K = 80
NCHUNK = EPW // K
RPT = 624
TAIL = N - NS * RPT
ZR = 24

_MESH = plsc.VectorSubcoreMesh(core_axis_name="c", subcore_axis_name="s")


def _fill_rows(ref, rows, d, value):
    splat = jnp.full((16,), value, dtype=jnp.float32)

    def body(i, carry):
        for jcol in range(d // 16):
            ref[i, pl.ds(jcol * 16, 16)] = splat
        return carry

    lax.fori_loop(0, rows, body, 0)


def _make_agg(d):
    @functools.partial(
        pl.kernel,
        out_type=jax.ShapeDtypeStruct((NC, N, d), jnp.float32),
        mesh=_MESH,
        compiler_params=pltpu.CompilerParams(use_tc_tiling_on_sc=False),
        scratch_types=[
            pltpu.VMEM((NCHUNK, K), jnp.int32),
            pltpu.VMEM((4, K), jnp.int32),
            pltpu.VMEM((4, K), jnp.int32),
            pltpu.VMEM((2, K, d), jnp.float32),
            pltpu.VMEM((ZR, d), jnp.float32),
            pltpu.VMEM_SHARED((N, d), jnp.float32),
            pltpu.SemaphoreType.DMA,
            pltpu.SemaphoreType.DMA,
            pltpu.SemaphoreType.DMA,
            pltpu.SemaphoreType.DMA,
            pltpu.SemaphoreType.DMA,
        ],
    )
    def agg(g_hbm, pk_hbm, out_hbm, pk_v, src_r, dst_r, rows_v, zero_v,
            acc, g0, g1, s0, s1, zsem):
        c = lax.axis_index("c")
        s = lax.axis_index("s")
        wid = c * NS + s
        gsem = (g0, g1)
        ssem = (s0, s1)

        _fill_rows(zero_v, ZR, d, 0.0)

        def zcopy(z):
            return pltpu.make_async_copy(
                zero_v, acc.at[pl.ds(s * RPT + z * ZR, ZR)], zsem)

        def zbody(z, carry):
            zcopy(z).start()
            return carry

        lax.fori_loop(0, RPT // ZR, zbody, 0)

        @pl.when(s == 0)
        def _():
            pltpu.sync_copy(zero_v.at[pl.ds(0, TAIL)],
                            acc.at[pl.ds(NS * RPT, TAIL)])

        pltpu.sync_copy(pk_hbm.at[wid], pk_v)

        def unpack(jj, q):
            for u in range(K // 16):
                w = pk_v[jj, pl.ds(u * 16, 16)]
                src_r[q, pl.ds(u * 16, 16)] = jnp.bitwise_and(w, 0xFFFF)
                dst_r[q, pl.ds(u * 16, 16)] = jnp.right_shift(w, 16)

        unpack(0, 0)
        unpack(1, 1)

        def zdrain(z, carry):
            zcopy(z).wait()
            return carry

        lax.fori_loop(0, RPT // ZR, zdrain, 0)
        plsc.subcore_barrier()

        def g_copy(rq, q):
            return pltpu.make_async_copy(
                g_hbm.at[src_r.at[q]], rows_v.at[rq], gsem[rq])

        def s_copy(rq, q):
            return pltpu.make_async_copy(
                rows_v.at[rq], acc.at[dst_r.at[q]], ssem[rq])

        def step(j, k, first=False, do_unpack=True, do_gather=True):
            rq, q = k % 2, k % 4
            g_copy(rq, q).wait()
            s_copy(rq, q).start(add=True)
            if not first:
                s_copy(1 - rq, (k + 3) % 4).wait()
            if do_unpack:
                unpack(j + 2, (k + 2) % 4)
            if do_gather:
                g_copy(1 - rq, (k + 1) % 4).start()

        g_copy(0, 0).start()
        step(0, 0, first=True)

        def quad(i, carry):
            j = 4 * i
            step(j + 1, 1)
            step(j + 2, 2)
            step(j + 3, 3)
            step(j + 4, 4)
            return carry

        lax.fori_loop(0, (NCHUNK - 5) // 4, quad, 0)

        step(NCHUNK - 4, 1)
        step(NCHUNK - 3, 2)
        step(NCHUNK - 2, 3, do_unpack=False)
        step(NCHUNK - 1, 0, do_unpack=False, do_gather=False)
        s_copy(0, 0).wait()
        plsc.subcore_barrier()

        pltpu.sync_copy(acc.at[pl.ds(s * RPT, RPT)],
                        out_hbm.at[c, pl.ds(s * RPT, RPT)])

        @pl.when(s == 0)
        def _():
            pltpu.sync_copy(acc.at[pl.ds(NS * RPT, TAIL)],
                            out_hbm.at[c, pl.ds(NS * RPT, TAIL)])

    return agg


_agg_hid = _make_agg(D_HID)
_agg_out = _make_agg(D_OUT)

_DEG_W = 16


@functools.partial(
    pl.kernel,
    out_type=jax.ShapeDtypeStruct((NC, N, _DEG_W), jnp.float32),
    mesh=_MESH,
    compiler_params=pltpu.CompilerParams(use_tc_tiling_on_sc=False),
    scratch_types=[
        pltpu.VMEM((NCHUNK, K), jnp.int32),
        pltpu.VMEM((NCHUNK, K), jnp.int32),
        pltpu.VMEM((K, _DEG_W), jnp.float32),
        pltpu.VMEM((ZR, _DEG_W), jnp.float32),
        pltpu.VMEM_SHARED((N, _DEG_W), jnp.float32),
        pltpu.SemaphoreType.DMA,
        pltpu.SemaphoreType.DMA,
        pltpu.SemaphoreType.DMA,
        pltpu.SemaphoreType.DMA,
        pltpu.SemaphoreType.DMA,
        pltpu.SemaphoreType.DMA,
    ],
)
def _deg_kernel(pk_hbm, out_hbm, pk_v, dst_v, ones_v, zero_v, acc,
                d0, d1, d2, d3, d4, zsem):
    c = lax.axis_index("c")
    s = lax.axis_index("s")
    wid = c * NS + s
    sems = (d0, d1, d2, d3, d4)

    _fill_rows(zero_v, ZR, _DEG_W, 0.0)
    _fill_rows(ones_v, K, _DEG_W, 1.0)

    def zcopy(z):
        return pltpu.make_async_copy(
            zero_v, acc.at[pl.ds(s * RPT + z * ZR, ZR)], zsem)

    def zbody(z, carry):
        zcopy(z).start()
        return carry

    lax.fori_loop(0, RPT // ZR, zbody, 0)

    @pl.when(s == 0)
    def _():
        pltpu.sync_copy(zero_v.at[pl.ds(0, TAIL)],
                        acc.at[pl.ds(NS * RPT, TAIL)])

    pltpu.sync_copy(pk_hbm.at[wid], pk_v)

    def unpack(jj, carry):
        for u in range(K // 16):
            w = pk_v[jj, pl.ds(u * 16, 16)]
            dst_v[jj, pl.ds(u * 16, 16)] = jnp.right_shift(w, 16)
        return carry

    lax.fori_loop(0, NCHUNK, unpack, 0)

    def zdrain(z, carry):
        zcopy(z).wait()
        return carry

    lax.fori_loop(0, RPT // ZR, zdrain, 0)
    plsc.subcore_barrier()

    def s_copy(j, sem):
        return pltpu.make_async_copy(ones_v, acc.at[dst_v.at[j]], sem)

    for k in range(5):
        s_copy(k, sems[k]).start(add=True)

    def ring(i, carry):
        j = 5 * i
        for k in range(5):
            s_copy(j + k, sems[k]).wait()
            s_copy(j + k + 5, sems[k]).start(add=True)
        return carry

    lax.fori_loop(0, NCHUNK // 5 - 1, ring, 0)

    for k in range(5):
        s_copy(NCHUNK - 5 + k, sems[k]).wait()

    plsc.subcore_barrier()

    pltpu.sync_copy(acc.at[pl.ds(s * RPT, RPT)],
                    out_hbm.at[c, pl.ds(s * RPT, RPT)])

    @pl.when(s == 0)
    def _():
        pltpu.sync_copy(acc.at[pl.ds(NS * RPT, TAIL)],
                        out_hbm.at[c, pl.ds(NS * RPT, TAIL)])


BLK = 1000


def _tc_a_body(x_ref, w1_ref, degp_ref, g_ref, dinv_ref):
    deg = degp_ref[0, :, 0:1] + degp_ref[1, :, 0:1] + 1.0
    dinv = lax.rsqrt(deg)
    h = jnp.dot(x_ref[...], w1_ref[...], preferred_element_type=jnp.float32)
    g_ref[...] = h * dinv
    dinv_ref[...] = jnp.broadcast_to(dinv, (BLK, _DEG_W))


def _tc_b_body(p_ref, g1_ref, dinv_ref, b1_ref, w2_ref, g2_ref):
    dinv = dinv_ref[:, 0:1]
    x2 = jnp.maximum((p_ref[0] + p_ref[1] + g1_ref[...]) * dinv + b1_ref[...],
                     0.0)
    g2_ref[...] = jnp.dot(x2, w2_ref[...],
                          preferred_element_type=jnp.float32) * dinv


def _tc_c_body(q_ref, g2_ref, dinv_ref, b2_ref, out_ref):
    dinv = dinv_ref[:, 0:1]
    z = (q_ref[0] + q_ref[1] + g2_ref[...]) * dinv + b2_ref[...]
    z = z - jnp.max(z, axis=1, keepdims=True)
    ez = jnp.exp(z)
    out_ref[...] = ez / jnp.sum(ez, axis=1, keepdims=True)


def _row_blocked(width):
    return pl.BlockSpec((BLK, width), lambda i: (i, 0))


def _pair_blocked(width):
    return pl.BlockSpec((2, BLK, width), lambda i: (0, i, 0))


def _full(shape):
    return pl.BlockSpec(shape, lambda i: tuple(0 for _ in shape))


def kernel(x, edge_index, W1, b1, W2, b2):
    src = edge_index[0].astype(jnp.int32)
    dst = edge_index[1].astype(jnp.int32)
    pk = jnp.bitwise_or(src, jnp.left_shift(dst, 16)).reshape(NW, NCHUNK, K)

    degp = _deg_kernel(pk)

    grid = N // BLK
    g1, dinv16 = pl.pallas_call(
        _tc_a_body,
        grid=(grid,),
        in_specs=[_row_blocked(D_IN), _full((D_IN, D_HID)),
                  _pair_blocked(_DEG_W)],
        out_specs=[_row_blocked(D_HID), _row_blocked(_DEG_W)],
        out_shape=[jax.ShapeDtypeStruct((N, D_HID), jnp.float32),
                   jax.ShapeDtypeStruct((N, _DEG_W), jnp.float32)],
    )(x, W1, degp)

    p = _agg_hid(g1, pk)

    g2 = pl.pallas_call(
        _tc_b_body,
        grid=(grid,),
        in_specs=[_pair_blocked(D_HID), _row_blocked(D_HID),
                  _row_blocked(_DEG_W), _full((1, D_HID)),
                  _full((D_HID, D_OUT))],
        out_specs=_row_blocked(D_OUT),
        out_shape=jax.ShapeDtypeStruct((N, D_OUT), jnp.float32),
    )(p, g1, dinv16, b1.reshape(1, D_HID), W2)

    q = _agg_out(g2, pk)

    out = pl.pallas_call(
        _tc_c_body,
        grid=(grid,),
        in_specs=[_pair_blocked(D_OUT), _row_blocked(D_OUT),
                  _row_blocked(_DEG_W), _full((1, D_OUT))],
        out_specs=_row_blocked(D_OUT),
        out_shape=jax.ShapeDtypeStruct((N, D_OUT), jnp.float32),
    )(q, g2, dinv16, b2.reshape(1, D_OUT))

    return out


# speedup vs baseline: 27.1468x; 1.0242x over previous
import functools

import jax
import jax.numpy as jnp
from jax import lax
from jax.experimental import pallas as pl
from jax.experimental.pallas import tpu as pltpu
from jax.experimental.pallas import tpu_sc as plsc

N = 10000
E = 320000
D_IN = 128
D_HID = 128
D_OUT = 64

NC = 2
NS = 16
NW = NC * NS
EPW = E // NW
K = 80
NCHUNK = EPW // K
RPT = 624
TAIL = N - NS * RPT
ZR = 24

_MESH = plsc.VectorSubcoreMesh(core_axis_name="c", subcore_axis_name="s")


def _fill_rows(ref, rows, d, value):
    splat = jnp.full((16,), value, dtype=jnp.float32)

    def body(i, carry):
        for jcol in range(d // 16):
            ref[i, pl.ds(jcol * 16, 16)] = splat
        return carry

    lax.fori_loop(0, rows, body, 0)


def _make_agg(d):
    @functools.partial(
        pl.kernel,
        out_type=jax.ShapeDtypeStruct((NC, N, d), jnp.float32),
        mesh=_MESH,
        compiler_params=pltpu.CompilerParams(use_tc_tiling_on_sc=False),
        scratch_types=[
            pltpu.VMEM((NCHUNK, K), jnp.int32),
            pltpu.VMEM((4, K), jnp.int32),
            pltpu.VMEM((4, K), jnp.int32),
            pltpu.VMEM((2, K, d), jnp.float32),
            pltpu.VMEM((ZR, d), jnp.float32),
            pltpu.VMEM_SHARED((N, d), jnp.float32),
            pltpu.SemaphoreType.DMA,
            pltpu.SemaphoreType.DMA,
            pltpu.SemaphoreType.DMA,
            pltpu.SemaphoreType.DMA,
            pltpu.SemaphoreType.DMA,
        ],
    )
    def agg(g_hbm, pk_hbm, out_hbm, pk_v, src_r, dst_r, rows_v, zero_v,
            acc, g0, g1, s0, s1, zsem):
        c = lax.axis_index("c")
        s = lax.axis_index("s")
        wid = c * NS + s
        gsem = (g0, g1)
        ssem = (s0, s1)

        _fill_rows(zero_v, ZR, d, 0.0)

        def zcopy(z):
            return pltpu.make_async_copy(
                zero_v, acc.at[pl.ds(s * RPT + z * ZR, ZR)], zsem)

        def zbody(z, carry):
            zcopy(z).start()
            return carry

        lax.fori_loop(0, RPT // ZR, zbody, 0)

        @pl.when(s == 0)
        def _():
            pltpu.sync_copy(zero_v.at[pl.ds(0, TAIL)],
                            acc.at[pl.ds(NS * RPT, TAIL)])

        pltpu.sync_copy(pk_hbm.at[wid], pk_v)

        def unpack(jj, q):
            for u in range(K // 16):
                w = pk_v[jj, pl.ds(u * 16, 16)]
                src_r[q, pl.ds(u * 16, 16)] = jnp.bitwise_and(w, 0xFFFF)
                dst_r[q, pl.ds(u * 16, 16)] = jnp.right_shift(w, 16)

        unpack(0, 0)
        unpack(1, 1)

        def zdrain(z, carry):
            zcopy(z).wait()
            return carry

        lax.fori_loop(0, RPT // ZR, zdrain, 0)
        plsc.subcore_barrier()

        def g_copy(rq, q):
            return pltpu.make_async_copy(
                g_hbm.at[src_r.at[q]], rows_v.at[rq], gsem[rq])

        def s_copy(rq, q):
            return pltpu.make_async_copy(
                rows_v.at[rq], acc.at[dst_r.at[q]], ssem[rq])

        def step(j, k, first=False, do_unpack=True, do_gather=True):
            rq, q = k % 2, k % 4
            g_copy(rq, q).wait()
            s_copy(rq, q).start(add=True)
            if not first:
                s_copy(1 - rq, (k + 3) % 4).wait()
            if do_unpack:
                unpack(j + 2, (k + 2) % 4)
            if do_gather:
                g_copy(1 - rq, (k + 1) % 4).start()

        g_copy(0, 0).start()
        step(0, 0, first=True)

        def quad(i, carry):
            j = 4 * i
            step(j + 1, 1)
            step(j + 2, 2)
            step(j + 3, 3)
            step(j + 4, 4)
            return carry

        lax.fori_loop(0, (NCHUNK - 5) // 4, quad, 0)

        step(NCHUNK - 4, 1)
        step(NCHUNK - 3, 2)
        step(NCHUNK - 2, 3, do_unpack=False)
        step(NCHUNK - 1, 0, do_unpack=False, do_gather=False)
        s_copy(0, 0).wait()
        plsc.subcore_barrier()

        pltpu.sync_copy(acc.at[pl.ds(s * RPT, RPT)],
                        out_hbm.at[c, pl.ds(s * RPT, RPT)])

        @pl.when(s == 0)
        def _():
            pltpu.sync_copy(acc.at[pl.ds(NS * RPT, TAIL)],
                            out_hbm.at[c, pl.ds(NS * RPT, TAIL)])

    return agg


_agg_hid = _make_agg(D_HID)
_agg_out = _make_agg(D_OUT)

_DEG_W = 16


@functools.partial(
    pl.kernel,
    out_type=[jax.ShapeDtypeStruct((NC, N, _DEG_W), jnp.float32),
              jax.ShapeDtypeStruct((NW, NCHUNK, K), jnp.int32)],
    mesh=_MESH,
    compiler_params=pltpu.CompilerParams(use_tc_tiling_on_sc=False),
    scratch_types=[
        pltpu.VMEM((EPW,), jnp.int32),
        pltpu.VMEM((EPW,), jnp.int32),
        pltpu.VMEM((NCHUNK, K), jnp.int32),
        pltpu.VMEM((NCHUNK, K), jnp.int32),
        pltpu.VMEM((K, _DEG_W), jnp.float32),
        pltpu.VMEM((ZR, _DEG_W), jnp.float32),
        pltpu.VMEM_SHARED((N, _DEG_W), jnp.float32),
        pltpu.SemaphoreType.DMA,
        pltpu.SemaphoreType.DMA,
        pltpu.SemaphoreType.DMA,
        pltpu.SemaphoreType.DMA,
        pltpu.SemaphoreType.DMA,
        pltpu.SemaphoreType.DMA,
        pltpu.SemaphoreType.DMA,
    ],
)
def _deg_kernel(ei_hbm, out_hbm, pk_hbm, src1_v, dst1_v, pk_v, dst_v,
                ones_v, zero_v, acc, d0, d1, d2, d3, d4, zsem, psem):
    c = lax.axis_index("c")
    s = lax.axis_index("s")
    wid = c * NS + s
    sems = (d0, d1, d2, d3, d4)

    _fill_rows(zero_v, ZR, _DEG_W, 0.0)
    _fill_rows(ones_v, K, _DEG_W, 1.0)

    def zcopy(z):
        return pltpu.make_async_copy(
            zero_v, acc.at[pl.ds(s * RPT + z * ZR, ZR)], zsem)

    def zbody(z, carry):
        zcopy(z).start()
        return carry

    lax.fori_loop(0, RPT // ZR, zbody, 0)

    @pl.when(s == 0)
    def _():
        pltpu.sync_copy(zero_v.at[pl.ds(0, TAIL)],
                        acc.at[pl.ds(NS * RPT, TAIL)])

    pltpu.sync_copy(ei_hbm.at[0, pl.ds(wid * EPW, EPW)], src1_v)
    pltpu.sync_copy(ei_hbm.at[1, pl.ds(wid * EPW, EPW)], dst1_v)

    def pack(jj, carry):
        for u in range(K // 16):
            col = u * 16
            sv = src1_v[pl.ds(jj * K + col, 16)]
            dv = dst1_v[pl.ds(jj * K + col, 16)]
            pk_v[jj, pl.ds(col, 16)] = jnp.bitwise_or(
                sv, jnp.left_shift(dv, 16))
            dst_v[jj, pl.ds(col, 16)] = dv
        return carry

    lax.fori_loop(0, NCHUNK, pack, 0)
    pk_out = pltpu.make_async_copy(pk_v, pk_hbm.at[wid], psem)
    pk_out.start()

    def zdrain(z, carry):
        zcopy(z).wait()
        return carry

    lax.fori_loop(0, RPT // ZR, zdrain, 0)
    plsc.subcore_barrier()

    def s_copy(j, sem):
        return pltpu.make_async_copy(ones_v, acc.at[dst_v.at[j]], sem)

    for k in range(5):
        s_copy(k, sems[k]).start(add=True)

    def ring(i, carry):
        j = 5 * i
        for k in range(5):
            s_copy(j + k, sems[k]).wait()
            s_copy(j + k + 5, sems[k]).start(add=True)
        return carry

    lax.fori_loop(0, NCHUNK // 5 - 1, ring, 0)

    for k in range(5):
        s_copy(NCHUNK - 5 + k, sems[k]).wait()

    pltpu.make_async_copy(pk_v, pk_hbm.at[wid], psem).wait()
    plsc.subcore_barrier()

    pltpu.sync_copy(acc.at[pl.ds(s * RPT, RPT)],
                    out_hbm.at[c, pl.ds(s * RPT, RPT)])

    @pl.when(s == 0)
    def _():
        pltpu.sync_copy(acc.at[pl.ds(NS * RPT, TAIL)],
                        out_hbm.at[c, pl.ds(NS * RPT, TAIL)])


BLK = 1000


def _tc_a_body(x_ref, w1_ref, degp_ref, g_ref, dinv_ref):
    deg = degp_ref[0, :, 0:1] + degp_ref[1, :, 0:1] + 1.0
    dinv = lax.rsqrt(deg)
    h = jnp.dot(x_ref[...], w1_ref[...], preferred_element_type=jnp.float32)
    g_ref[...] = h * dinv
    dinv_ref[...] = jnp.broadcast_to(dinv, (BLK, _DEG_W))


def _tc_b_body(p_ref, g1_ref, dinv_ref, b1_ref, w2_ref, g2_ref):
    dinv = dinv_ref[:, 0:1]
    x2 = jnp.maximum((p_ref[0] + p_ref[1] + g1_ref[...]) * dinv + b1_ref[...],
                     0.0)
    g2_ref[...] = jnp.dot(x2, w2_ref[...],
                          preferred_element_type=jnp.float32) * dinv


def _tc_c_body(q_ref, g2_ref, dinv_ref, b2_ref, out_ref):
    dinv = dinv_ref[:, 0:1]
    z = (q_ref[0] + q_ref[1] + g2_ref[...]) * dinv + b2_ref[...]
    z = z - jnp.max(z, axis=1, keepdims=True)
    ez = jnp.exp(z)
    out_ref[...] = ez / jnp.sum(ez, axis=1, keepdims=True)


def _row_blocked(width):
    return pl.BlockSpec((BLK, width), lambda i: (i, 0))


def _pair_blocked(width):
    return pl.BlockSpec((2, BLK, width), lambda i: (0, i, 0))


def _full(shape):
    return pl.BlockSpec(shape, lambda i: tuple(0 for _ in shape))


def kernel(x, edge_index, W1, b1, W2, b2):
    degp, pk = _deg_kernel(edge_index.astype(jnp.int32))

    grid = N // BLK
    g1, dinv16 = pl.pallas_call(
        _tc_a_body,
        grid=(grid,),
        in_specs=[_row_blocked(D_IN), _full((D_IN, D_HID)),
                  _pair_blocked(_DEG_W)],
        out_specs=[_row_blocked(D_HID), _row_blocked(_DEG_W)],
        out_shape=[jax.ShapeDtypeStruct((N, D_HID), jnp.float32),
                   jax.ShapeDtypeStruct((N, _DEG_W), jnp.float32)],
    )(x, W1, degp)

    p = _agg_hid(g1, pk)

    g2 = pl.pallas_call(
        _tc_b_body,
        grid=(grid,),
        in_specs=[_pair_blocked(D_HID), _row_blocked(D_HID),
                  _row_blocked(_DEG_W), _full((1, D_HID)),
                  _full((D_HID, D_OUT))],
        out_specs=_row_blocked(D_OUT),
        out_shape=jax.ShapeDtypeStruct((N, D_OUT), jnp.float32),
    )(p, g1, dinv16, b1.reshape(1, D_HID), W2)

    q = _agg_out(g2, pk)

    out = pl.pallas_call(
        _tc_c_body,
        grid=(grid,),
        in_specs=[_pair_blocked(D_OUT), _row_blocked(D_OUT),
                  _row_blocked(_DEG_W), _full((1, D_OUT))],
        out_specs=_row_blocked(D_OUT),
        out_shape=jax.ShapeDtypeStruct((N, D_OUT), jnp.float32),
    )(q, g2, dinv16, b2.reshape(1, D_OUT))

    return out


# speedup vs baseline: 30.3488x; 1.1180x over previous
import functools

import jax
import jax.numpy as jnp
from jax import lax
from jax.experimental import pallas as pl
from jax.experimental.pallas import tpu as pltpu
from jax.experimental.pallas import tpu_sc as plsc

N = 10000
E = 320000
D_IN = 128
D_HID = 128
D_OUT = 64

NC = 2
NS = 16
NW = NC * NS
EPW = E // NW
K = 80
NCHUNK = EPW // K
RPT = 624
TAIL = N - NS * RPT
ZR = 24

_MESH = plsc.VectorSubcoreMesh(core_axis_name="c", subcore_axis_name="s")


def _fill_rows(ref, rows, d, value):
    splat = jnp.full((16,), value, dtype=jnp.float32)

    def body(i, carry):
        for jcol in range(d // 16):
            ref[i, pl.ds(jcol * 16, 16)] = splat
        return carry

    lax.fori_loop(0, rows, body, 0)


def _make_agg(d):
    scratch = [
        pltpu.VMEM((NCHUNK, K), jnp.int32),
        pltpu.VMEM((4, K), jnp.int32),
        pltpu.VMEM((4, K), jnp.int32),
        pltpu.VMEM((2, K, d), jnp.float32),
        pltpu.VMEM((ZR, d), jnp.float32),
        pltpu.VMEM_SHARED((N, d), jnp.float32),
        pltpu.SemaphoreType.DMA,
        pltpu.SemaphoreType.DMA,
        pltpu.SemaphoreType.DMA,
        pltpu.SemaphoreType.DMA,
        pltpu.SemaphoreType.DMA,
    ]
    spmem_table = d <= 64
    if spmem_table:
        scratch.append(pltpu.VMEM_SHARED((N, d), jnp.float32))

    @functools.partial(
        pl.kernel,
        out_type=jax.ShapeDtypeStruct((NC, N, d), jnp.float32),
        mesh=_MESH,
        compiler_params=pltpu.CompilerParams(use_tc_tiling_on_sc=False),
        scratch_types=scratch,
    )
    def agg(g_hbm, pk_hbm, out_hbm, pk_v, src_r, dst_r, rows_v, zero_v,
            acc, g0, g1, s0, s1, zsem, *maybe_table):
        table = maybe_table[0] if spmem_table else g_hbm
        c = lax.axis_index("c")
        s = lax.axis_index("s")
        wid = c * NS + s
        gsem = (g0, g1)
        ssem = (s0, s1)

        _fill_rows(zero_v, ZR, d, 0.0)

        def zcopy(z):
            return pltpu.make_async_copy(
                zero_v, acc.at[pl.ds(s * RPT + z * ZR, ZR)], zsem)

        def zbody(z, carry):
            zcopy(z).start()
            return carry

        lax.fori_loop(0, RPT // ZR, zbody, 0)

        @pl.when(s == 0)
        def _():
            pltpu.sync_copy(zero_v.at[pl.ds(0, TAIL)],
                            acc.at[pl.ds(NS * RPT, TAIL)])

        pltpu.sync_copy(pk_hbm.at[wid], pk_v)

        def unpack(jj, q):
            for u in range(K // 16):
                w = pk_v[jj, pl.ds(u * 16, 16)]
                src_r[q, pl.ds(u * 16, 16)] = jnp.bitwise_and(w, 0xFFFF)
                dst_r[q, pl.ds(u * 16, 16)] = jnp.right_shift(w, 16)

        unpack(0, 0)
        unpack(1, 1)

        def zdrain(z, carry):
            zcopy(z).wait()
            return carry

        lax.fori_loop(0, RPT // ZR, zdrain, 0)
        plsc.subcore_barrier()

        if spmem_table:
            pltpu.sync_copy(g_hbm.at[pl.ds(s * RPT, RPT)],
                            table.at[pl.ds(s * RPT, RPT)])

            @pl.when(s == 0)
            def _():
                pltpu.sync_copy(g_hbm.at[pl.ds(NS * RPT, TAIL)],
                                table.at[pl.ds(NS * RPT, TAIL)])

            plsc.subcore_barrier()

        def g_copy(rq, q):
            return pltpu.make_async_copy(
                table.at[src_r.at[q]], rows_v.at[rq], gsem[rq])

        def s_copy(rq, q):
            return pltpu.make_async_copy(
                rows_v.at[rq], acc.at[dst_r.at[q]], ssem[rq])

        def step(j, k, first=False, do_unpack=True, do_gather=True):
            rq, q = k % 2, k % 4
            g_copy(rq, q).wait()
            s_copy(rq, q).start(add=True)
            if not first:
                s_copy(1 - rq, (k + 3) % 4).wait()
            if do_unpack:
                unpack(j + 2, (k + 2) % 4)
            if do_gather:
                g_copy(1 - rq, (k + 1) % 4).start()

        g_copy(0, 0).start()
        step(0, 0, first=True)

        def quad(i, carry):
            j = 4 * i
            step(j + 1, 1)
            step(j + 2, 2)
            step(j + 3, 3)
            step(j + 4, 4)
            return carry

        lax.fori_loop(0, (NCHUNK - 5) // 4, quad, 0)

        step(NCHUNK - 4, 1)
        step(NCHUNK - 3, 2)
        step(NCHUNK - 2, 3, do_unpack=False)
        step(NCHUNK - 1, 0, do_unpack=False, do_gather=False)
        s_copy(0, 0).wait()
        plsc.subcore_barrier()

        pltpu.sync_copy(acc.at[pl.ds(s * RPT, RPT)],
                        out_hbm.at[c, pl.ds(s * RPT, RPT)])

        @pl.when(s == 0)
        def _():
            pltpu.sync_copy(acc.at[pl.ds(NS * RPT, TAIL)],
                            out_hbm.at[c, pl.ds(NS * RPT, TAIL)])

    return agg


_agg_hid = _make_agg(D_HID)
_agg_out = _make_agg(D_OUT)

_DEG_W = 16


@functools.partial(
    pl.kernel,
    out_type=[jax.ShapeDtypeStruct((NC, N, _DEG_W), jnp.float32),
              jax.ShapeDtypeStruct((NW, NCHUNK, K), jnp.int32)],
    mesh=_MESH,
    compiler_params=pltpu.CompilerParams(use_tc_tiling_on_sc=False),
    scratch_types=[
        pltpu.VMEM((EPW,), jnp.int32),
        pltpu.VMEM((EPW,), jnp.int32),
        pltpu.VMEM((NCHUNK, K), jnp.int32),
        pltpu.VMEM((NCHUNK, K), jnp.int32),
        pltpu.VMEM((K, _DEG_W), jnp.float32),
        pltpu.VMEM((ZR, _DEG_W), jnp.float32),
        pltpu.VMEM_SHARED((N, _DEG_W), jnp.float32),
        pltpu.SemaphoreType.DMA,
        pltpu.SemaphoreType.DMA,
        pltpu.SemaphoreType.DMA,
        pltpu.SemaphoreType.DMA,
        pltpu.SemaphoreType.DMA,
        pltpu.SemaphoreType.DMA,
        pltpu.SemaphoreType.DMA,
    ],
)
def _deg_kernel(ei_hbm, out_hbm, pk_hbm, src1_v, dst1_v, pk_v, dst_v,
                ones_v, zero_v, acc, d0, d1, d2, d3, d4, zsem, psem):
    c = lax.axis_index("c")
    s = lax.axis_index("s")
    wid = c * NS + s
    sems = (d0, d1, d2, d3, d4)

    _fill_rows(zero_v, ZR, _DEG_W, 0.0)
    _fill_rows(ones_v, K, _DEG_W, 1.0)

    def zcopy(z):
        return pltpu.make_async_copy(
            zero_v, acc.at[pl.ds(s * RPT + z * ZR, ZR)], zsem)

    def zbody(z, carry):
        zcopy(z).start()
        return carry

    lax.fori_loop(0, RPT // ZR, zbody, 0)

    @pl.when(s == 0)
    def _():
        pltpu.sync_copy(zero_v.at[pl.ds(0, TAIL)],
                        acc.at[pl.ds(NS * RPT, TAIL)])

    pltpu.sync_copy(ei_hbm.at[0, pl.ds(wid * EPW, EPW)], src1_v)
    pltpu.sync_copy(ei_hbm.at[1, pl.ds(wid * EPW, EPW)], dst1_v)

    def pack(jj, carry):
        for u in range(K // 16):
            col = u * 16
            sv = src1_v[pl.ds(jj * K + col, 16)]
            dv = dst1_v[pl.ds(jj * K + col, 16)]
            pk_v[jj, pl.ds(col, 16)] = jnp.bitwise_or(
                sv, jnp.left_shift(dv, 16))
            dst_v[jj, pl.ds(col, 16)] = dv
        return carry

    lax.fori_loop(0, NCHUNK, pack, 0)
    pk_out = pltpu.make_async_copy(pk_v, pk_hbm.at[wid], psem)
    pk_out.start()

    def zdrain(z, carry):
        zcopy(z).wait()
        return carry

    lax.fori_loop(0, RPT // ZR, zdrain, 0)
    plsc.subcore_barrier()

    def s_copy(j, sem):
        return pltpu.make_async_copy(ones_v, acc.at[dst_v.at[j]], sem)

    for k in range(5):
        s_copy(k, sems[k]).start(add=True)

    def ring(i, carry):
        j = 5 * i
        for k in range(5):
            s_copy(j + k, sems[k]).wait()
            s_copy(j + k + 5, sems[k]).start(add=True)
        return carry

    lax.fori_loop(0, NCHUNK // 5 - 1, ring, 0)

    for k in range(5):
        s_copy(NCHUNK - 5 + k, sems[k]).wait()

    pltpu.make_async_copy(pk_v, pk_hbm.at[wid], psem).wait()
    plsc.subcore_barrier()

    pltpu.sync_copy(acc.at[pl.ds(s * RPT, RPT)],
                    out_hbm.at[c, pl.ds(s * RPT, RPT)])

    @pl.when(s == 0)
    def _():
        pltpu.sync_copy(acc.at[pl.ds(NS * RPT, TAIL)],
                        out_hbm.at[c, pl.ds(NS * RPT, TAIL)])


BLK = 1000


def _tc_a_body(x_ref, w1_ref, degp_ref, g_ref, dinv_ref):
    deg = degp_ref[0, :, 0:1] + degp_ref[1, :, 0:1] + 1.0
    dinv = lax.rsqrt(deg)
    h = jnp.dot(x_ref[...], w1_ref[...], preferred_element_type=jnp.float32)
    g_ref[...] = h * dinv
    dinv_ref[...] = jnp.broadcast_to(dinv, (BLK, _DEG_W))


def _tc_b_body(p_ref, g1_ref, dinv_ref, b1_ref, w2_ref, g2_ref):
    dinv = dinv_ref[:, 0:1]
    x2 = jnp.maximum((p_ref[0] + p_ref[1] + g1_ref[...]) * dinv + b1_ref[...],
                     0.0)
    g2_ref[...] = jnp.dot(x2, w2_ref[...],
                          preferred_element_type=jnp.float32) * dinv


def _tc_c_body(q_ref, g2_ref, dinv_ref, b2_ref, out_ref):
    dinv = dinv_ref[:, 0:1]
    z = (q_ref[0] + q_ref[1] + g2_ref[...]) * dinv + b2_ref[...]
    z = z - jnp.max(z, axis=1, keepdims=True)
    ez = jnp.exp(z)
    out_ref[...] = ez / jnp.sum(ez, axis=1, keepdims=True)


def _row_blocked(width):
    return pl.BlockSpec((BLK, width), lambda i: (i, 0))


def _pair_blocked(width):
    return pl.BlockSpec((2, BLK, width), lambda i: (0, i, 0))


def _full(shape):
    return pl.BlockSpec(shape, lambda i: tuple(0 for _ in shape))


def kernel(x, edge_index, W1, b1, W2, b2):
    degp, pk = _deg_kernel(edge_index.astype(jnp.int32))

    grid = N // BLK
    g1, dinv16 = pl.pallas_call(
        _tc_a_body,
        grid=(grid,),
        in_specs=[_row_blocked(D_IN), _full((D_IN, D_HID)),
                  _pair_blocked(_DEG_W)],
        out_specs=[_row_blocked(D_HID), _row_blocked(_DEG_W)],
        out_shape=[jax.ShapeDtypeStruct((N, D_HID), jnp.float32),
                   jax.ShapeDtypeStruct((N, _DEG_W), jnp.float32)],
    )(x, W1, degp)

    p = _agg_hid(g1, pk)

    g2 = pl.pallas_call(
        _tc_b_body,
        grid=(grid,),
        in_specs=[_pair_blocked(D_HID), _row_blocked(D_HID),
                  _row_blocked(_DEG_W), _full((1, D_HID)),
                  _full((D_HID, D_OUT))],
        out_specs=_row_blocked(D_OUT),
        out_shape=jax.ShapeDtypeStruct((N, D_OUT), jnp.float32),
    )(p, g1, dinv16, b1.reshape(1, D_HID), W2)

    q = _agg_out(g2, pk)

    out = pl.pallas_call(
        _tc_c_body,
        grid=(grid,),
        in_specs=[_pair_blocked(D_OUT), _row_blocked(D_OUT),
                  _row_blocked(_DEG_W), _full((1, D_OUT))],
        out_specs=_row_blocked(D_OUT),
        out_shape=jax.ShapeDtypeStruct((N, D_OUT), jnp.float32),
    )(q, g2, dinv16, b2.reshape(1, D_OUT))

    return out


# speedup vs baseline: 34.2565x; 1.1288x over previous
import functools

import jax
import jax.numpy as jnp
from jax import lax
from jax.experimental import pallas as pl
from jax.experimental.pallas import tpu as pltpu
from jax.experimental.pallas import tpu_sc as plsc

N = 10000
E = 320000
D_IN = 128
D_HID = 128
D_OUT = 64

NC = 2
NS = 16
NW = NC * NS
EPW = E // NW
K = 80
NCHUNK = EPW // K
RPT = 624
TAIL = N - NS * RPT
ZR = 24

_MESH = plsc.VectorSubcoreMesh(core_axis_name="c", subcore_axis_name="s")


def _fill_rows(ref, rows, d, value):
    splat = jnp.full((16,), value, dtype=jnp.float32)

    def body(i, carry):
        for jcol in range(d // 16):
            ref[i, pl.ds(jcol * 16, 16)] = splat
        return carry

    lax.fori_loop(0, rows, body, 0)


def _make_agg(d):
    scratch = [
        pltpu.VMEM((NCHUNK, K), jnp.int32),
        pltpu.VMEM((4, K), jnp.int32),
        pltpu.VMEM((4, K), jnp.int32),
        pltpu.VMEM((2, K, d), jnp.float32),
        pltpu.VMEM((ZR, d), jnp.float32),
        pltpu.VMEM_SHARED((N, d), jnp.float32),
        pltpu.SemaphoreType.DMA,
        pltpu.SemaphoreType.DMA,
        pltpu.SemaphoreType.DMA,
        pltpu.SemaphoreType.DMA,
        pltpu.SemaphoreType.DMA,
    ]
    spmem_table = d <= 64
    if spmem_table:
        scratch.append(pltpu.VMEM_SHARED((N, d), jnp.float32))

    @functools.partial(
        pl.kernel,
        out_type=jax.ShapeDtypeStruct((NC, N, d), jnp.float32),
        mesh=_MESH,
        compiler_params=pltpu.CompilerParams(use_tc_tiling_on_sc=False),
        scratch_types=scratch,
    )
    def agg(g_hbm, pk_hbm, out_hbm, pk_v, src_r, dst_r, rows_v, zero_v,
            acc, g0, g1, s0, s1, zsem, *maybe_table):
        table = maybe_table[0] if spmem_table else g_hbm
        c = lax.axis_index("c")
        s = lax.axis_index("s")
        wid = c * NS + s
        gsem = (g0, g1)
        ssem = (s0, s1)

        _fill_rows(zero_v, ZR, d, 0.0)

        def zcopy(z):
            return pltpu.make_async_copy(
                zero_v, acc.at[pl.ds(s * RPT + z * ZR, ZR)], zsem)

        def zbody(z, carry):
            zcopy(z).start()
            return carry

        lax.fori_loop(0, RPT // ZR, zbody, 0)

        @pl.when(s == 0)
        def _():
            pltpu.sync_copy(zero_v.at[pl.ds(0, TAIL)],
                            acc.at[pl.ds(NS * RPT, TAIL)])

        pltpu.sync_copy(pk_hbm.at[wid], pk_v)

        def unpack(jj, q):
            for u in range(K // 16):
                w = pk_v[jj, pl.ds(u * 16, 16)]
                src_r[q, pl.ds(u * 16, 16)] = jnp.bitwise_and(w, 0xFFFF)
                dst_r[q, pl.ds(u * 16, 16)] = jnp.right_shift(w, 16)

        unpack(0, 0)

        def zdrain(z, carry):
            zcopy(z).wait()
            return carry

        lax.fori_loop(0, RPT // ZR, zdrain, 0)
        plsc.subcore_barrier()

        if spmem_table:
            pltpu.sync_copy(g_hbm.at[pl.ds(s * RPT, RPT)],
                            table.at[pl.ds(s * RPT, RPT)])

            @pl.when(s == 0)
            def _():
                pltpu.sync_copy(g_hbm.at[pl.ds(NS * RPT, TAIL)],
                                table.at[pl.ds(NS * RPT, TAIL)])

            plsc.subcore_barrier()

        def g_copy(rq, q):
            return pltpu.make_async_copy(
                table.at[src_r.at[q]], rows_v.at[rq], gsem[rq])

        def s_copy(rq, q):
            return pltpu.make_async_copy(
                rows_v.at[rq], acc.at[dst_r.at[q]], ssem[rq])

        def step(j, k, first=False, do_next=True):
            rq, q = k % 2, k % 4
            if not first:
                s_copy(1 - rq, (k + 3) % 4).wait()
            if do_next:
                unpack(j + 1, (k + 1) % 4)
                g_copy(1 - rq, (k + 1) % 4).start()
            g_copy(rq, q).wait()
            s_copy(rq, q).start(add=True)

        g_copy(0, 0).start()
        step(0, 0, first=True)

        def quad(i, carry):
            j = 4 * i
            step(j + 1, 1)
            step(j + 2, 2)
            step(j + 3, 3)
            step(j + 4, 4)
            return carry

        lax.fori_loop(0, (NCHUNK - 5) // 4, quad, 0)

        step(NCHUNK - 4, 1)
        step(NCHUNK - 3, 2)
        step(NCHUNK - 2, 3)
        step(NCHUNK - 1, 0, do_next=False)
        s_copy(0, 0).wait()
        plsc.subcore_barrier()

        pltpu.sync_copy(acc.at[pl.ds(s * RPT, RPT)],
                        out_hbm.at[c, pl.ds(s * RPT, RPT)])

        @pl.when(s == 0)
        def _():
            pltpu.sync_copy(acc.at[pl.ds(NS * RPT, TAIL)],
                            out_hbm.at[c, pl.ds(NS * RPT, TAIL)])

    return agg


_agg_hid = _make_agg(D_HID)
_agg_out = _make_agg(D_OUT)

_DEG_W = 16


@functools.partial(
    pl.kernel,
    out_type=[jax.ShapeDtypeStruct((NC, N, _DEG_W), jnp.float32),
              jax.ShapeDtypeStruct((NW, NCHUNK, K), jnp.int32)],
    mesh=_MESH,
    compiler_params=pltpu.CompilerParams(use_tc_tiling_on_sc=False),
    scratch_types=[
        pltpu.VMEM((EPW,), jnp.int32),
        pltpu.VMEM((EPW,), jnp.int32),
        pltpu.VMEM((NCHUNK, K), jnp.int32),
        pltpu.VMEM((NCHUNK, K), jnp.int32),
        pltpu.VMEM((K, _DEG_W), jnp.float32),
        pltpu.VMEM((ZR, _DEG_W), jnp.float32),
        pltpu.VMEM_SHARED((N, _DEG_W), jnp.float32),
        pltpu.SemaphoreType.DMA,
        pltpu.SemaphoreType.DMA,
        pltpu.SemaphoreType.DMA,
        pltpu.SemaphoreType.DMA,
        pltpu.SemaphoreType.DMA,
        pltpu.SemaphoreType.DMA,
        pltpu.SemaphoreType.DMA,
    ],
)
def _deg_kernel(ei_hbm, out_hbm, pk_hbm, src1_v, dst1_v, pk_v, dst_v,
                ones_v, zero_v, acc, d0, d1, d2, d3, d4, zsem, psem):
    c = lax.axis_index("c")
    s = lax.axis_index("s")
    wid = c * NS + s
    sems = (d0, d1, d2, d3, d4)

    _fill_rows(zero_v, ZR, _DEG_W, 0.0)
    _fill_rows(ones_v, K, _DEG_W, 1.0)

    def zcopy(z):
        return pltpu.make_async_copy(
            zero_v, acc.at[pl.ds(s * RPT + z * ZR, ZR)], zsem)

    def zbody(z, carry):
        zcopy(z).start()
        return carry

    lax.fori_loop(0, RPT // ZR, zbody, 0)

    @pl.when(s == 0)
    def _():
        pltpu.sync_copy(zero_v.at[pl.ds(0, TAIL)],
                        acc.at[pl.ds(NS * RPT, TAIL)])

    pltpu.sync_copy(ei_hbm.at[0, pl.ds(wid * EPW, EPW)], src1_v)
    pltpu.sync_copy(ei_hbm.at[1, pl.ds(wid * EPW, EPW)], dst1_v)

    def pack(jj, carry):
        for u in range(K // 16):
            col = u * 16
            sv = src1_v[pl.ds(jj * K + col, 16)]
            dv = dst1_v[pl.ds(jj * K + col, 16)]
            pk_v[jj, pl.ds(col, 16)] = jnp.bitwise_or(
                sv, jnp.left_shift(dv, 16))
            dst_v[jj, pl.ds(col, 16)] = dv
        return carry

    lax.fori_loop(0, NCHUNK, pack, 0)
    pk_out = pltpu.make_async_copy(pk_v, pk_hbm.at[wid], psem)
    pk_out.start()

    def zdrain(z, carry):
        zcopy(z).wait()
        return carry

    lax.fori_loop(0, RPT // ZR, zdrain, 0)
    plsc.subcore_barrier()

    def s_copy(j, sem):
        return pltpu.make_async_copy(ones_v, acc.at[dst_v.at[j]], sem)

    for k in range(5):
        s_copy(k, sems[k]).start(add=True)

    def ring(i, carry):
        j = 5 * i
        for k in range(5):
            s_copy(j + k, sems[k]).wait()
            s_copy(j + k + 5, sems[k]).start(add=True)
        return carry

    lax.fori_loop(0, NCHUNK // 5 - 1, ring, 0)

    for k in range(5):
        s_copy(NCHUNK - 5 + k, sems[k]).wait()

    pltpu.make_async_copy(pk_v, pk_hbm.at[wid], psem).wait()
    plsc.subcore_barrier()

    pltpu.sync_copy(acc.at[pl.ds(s * RPT, RPT)],
                    out_hbm.at[c, pl.ds(s * RPT, RPT)])

    @pl.when(s == 0)
    def _():
        pltpu.sync_copy(acc.at[pl.ds(NS * RPT, TAIL)],
                        out_hbm.at[c, pl.ds(NS * RPT, TAIL)])


BLK = 1000


def _tc_a_body(x_ref, w1_ref, degp_ref, g_ref, dinv_ref):
    deg = degp_ref[0, :, 0:1] + degp_ref[1, :, 0:1] + 1.0
    dinv = lax.rsqrt(deg)
    h = jnp.dot(x_ref[...], w1_ref[...], preferred_element_type=jnp.float32)
    g_ref[...] = h * dinv
    dinv_ref[...] = jnp.broadcast_to(dinv, (BLK, _DEG_W))


def _tc_b_body(p_ref, g1_ref, dinv_ref, b1_ref, w2_ref, g2_ref):
    dinv = dinv_ref[:, 0:1]
    x2 = jnp.maximum((p_ref[0] + p_ref[1] + g1_ref[...]) * dinv + b1_ref[...],
                     0.0)
    g2_ref[...] = jnp.dot(x2, w2_ref[...],
                          preferred_element_type=jnp.float32) * dinv


def _tc_c_body(q_ref, g2_ref, dinv_ref, b2_ref, out_ref):
    dinv = dinv_ref[:, 0:1]
    z = (q_ref[0] + q_ref[1] + g2_ref[...]) * dinv + b2_ref[...]
    z = z - jnp.max(z, axis=1, keepdims=True)
    ez = jnp.exp(z)
    out_ref[...] = ez / jnp.sum(ez, axis=1, keepdims=True)


def _row_blocked(width):
    return pl.BlockSpec((BLK, width), lambda i: (i, 0))


def _pair_blocked(width):
    return pl.BlockSpec((2, BLK, width), lambda i: (0, i, 0))


def _full(shape):
    return pl.BlockSpec(shape, lambda i: tuple(0 for _ in shape))


def kernel(x, edge_index, W1, b1, W2, b2):
    degp, pk = _deg_kernel(edge_index.astype(jnp.int32))

    grid = N // BLK
    g1, dinv16 = pl.pallas_call(
        _tc_a_body,
        grid=(grid,),
        in_specs=[_row_blocked(D_IN), _full((D_IN, D_HID)),
                  _pair_blocked(_DEG_W)],
        out_specs=[_row_blocked(D_HID), _row_blocked(_DEG_W)],
        out_shape=[jax.ShapeDtypeStruct((N, D_HID), jnp.float32),
                   jax.ShapeDtypeStruct((N, _DEG_W), jnp.float32)],
    )(x, W1, degp)

    p = _agg_hid(g1, pk)

    g2 = pl.pallas_call(
        _tc_b_body,
        grid=(grid,),
        in_specs=[_pair_blocked(D_HID), _row_blocked(D_HID),
                  _row_blocked(_DEG_W), _full((1, D_HID)),
                  _full((D_HID, D_OUT))],
        out_specs=_row_blocked(D_OUT),
        out_shape=jax.ShapeDtypeStruct((N, D_OUT), jnp.float32),
    )(p, g1, dinv16, b1.reshape(1, D_HID), W2)

    q = _agg_out(g2, pk)

    out = pl.pallas_call(
        _tc_c_body,
        grid=(grid,),
        in_specs=[_pair_blocked(D_OUT), _row_blocked(D_OUT),
                  _row_blocked(_DEG_W), _full((1, D_OUT))],
        out_specs=_row_blocked(D_OUT),
        out_shape=jax.ShapeDtypeStruct((N, D_OUT), jnp.float32),
    )(q, g2, dinv16, b2.reshape(1, D_OUT))

    return out
